# Initial kernel scaffold; baseline (speedup 1.0000x reference)
import jax
import jax.numpy as jnp
from jax.experimental import pallas as pl


def kernel(points, neighbors, feats, Wq, bq, Wk, bk, Wv, bv, Wp, bp, g_p, be_p, Ww, bw, g_w, be_w, g_o, be_o):
    raise NotImplementedError("write your pallas kernel here")


# speedup vs baseline: 1.1205x; 1.1205x over previous
import functools

import jax
import jax.numpy as jnp
from jax.experimental import pallas as pl
from jax.experimental.pallas import tpu as pltpu
from jax.experimental.pallas import tpu_sc as plsc

N = 10000
K = 32
C = 128
NK = N * K
PW = 16
BQ = 200
NBLK = N // BQ
GATHER_WIN = 128
NKPAD = 327680
EPS = 1e-5


def _sc_gather(table, idx):
    mesh = plsc.VectorSubcoreMesh(core_axis_name="c", subcore_axis_name="s")

    @functools.partial(
        pl.kernel,
        out_type=jax.ShapeDtypeStruct((NKPAD, C), table.dtype),
        mesh=mesh,
    )
    def knl(tab_hbm, idx_hbm, out_hbm):
        def body(i_vmem, o_vmem):
            pltpu.sync_copy(tab_hbm.at[i_vmem.at[0]], o_vmem)

        pltpu.emit_pipeline(
            body,
            grid=(NKPAD // GATHER_WIN,),
            in_specs=[pl.BlockSpec((1, GATHER_WIN), lambda i: (0, i))],
            out_specs=[pl.BlockSpec((GATHER_WIN, C), lambda i: (i, 0))],
            core_axis_name=("c", "s"),
            dimension_semantics=(pltpu.PARALLEL,),
        )(idx_hbm, out_hbm)

    return knl(table, idx)


def _qkv_body(f_ref, p_ref, wq_ref, bq_ref, wk_ref, bk_ref, wv_ref, bv_ref,
              wp_ref, bp_ref, q_ref, k_ref, v_ref, xp_ref):
    f = f_ref[...]
    q_ref[...] = jnp.dot(f, wq_ref[...], preferred_element_type=jnp.float32) + bq_ref[...]
    k_ref[...] = jnp.dot(f, wk_ref[...], preferred_element_type=jnp.float32) + bk_ref[...]
    v_ref[...] = jnp.dot(f, wv_ref[...], preferred_element_type=jnp.float32) + bv_ref[...]
    xp_ref[...] = jnp.dot(p_ref[...], wp_ref[...],
                          preferred_element_type=jnp.float32) + bp_ref[...]


def _qkv(feats, p16, wq, bq, wk, bk, wv, bv, wp16, bp):
    s = jax.ShapeDtypeStruct((N, C), jnp.float32)
    return pl.pallas_call(_qkv_body, out_shape=(s, s, s, s))(
        feats, p16, wq, bq, wk, bk, wv, bv, wp16, bp)


def _stats_update(s_ref, x, first):
    upd = jnp.concatenate(
        [jnp.sum(x, axis=0, keepdims=True),
         jnp.sum(x * x, axis=0, keepdims=True),
         jnp.zeros((6, C), jnp.float32)], axis=0)

    @pl.when(first)
    def _():
        s_ref[...] = jnp.zeros_like(s_ref)

    s_ref[...] += upd


def _affine(s_ref, count, g, be):
    mean = s_ref[0:1, :] / count
    var = s_ref[1:2, :] / count - mean * mean
    scale = jax.lax.rsqrt(var + EPS) * g
    return scale, be - mean * scale


def _pstats_body(xp_ref, s_ref):
    x = xp_ref[...].reshape(K * BQ, C)
    _stats_update(s_ref, x, pl.program_id(0) == 0)


def _pstats(xp3):
    return pl.pallas_call(
        _pstats_body,
        grid=(NBLK,),
        in_specs=[pl.BlockSpec((K, BQ, C), lambda i: (0, i, 0))],
        out_specs=pl.BlockSpec((8, C), lambda i: (0, 0)),
        out_shape=jax.ShapeDtypeStruct((8, C), jnp.float32),
    )(xp3)


def _wpre(k_ref, xp_ref, q_ref, sp_ref, gp_ref, bep_ref, ww_ref, bw_ref):
    sc, sh = _affine(sp_ref, float(NK), gp_ref[...], bep_ref[...])
    pf = jnp.maximum(xp_ref[...].reshape(K * BQ, C) * sc + sh, 0.0)
    kq = (k_ref[...] * q_ref[...][None]).reshape(K * BQ, C)
    wpre = jnp.dot(kq + pf, ww_ref[...],
                   preferred_element_type=jnp.float32) + bw_ref[...]
    return pf, wpre


def _wstats_body(k_ref, xp_ref, q_ref, sp_ref, gp_ref, bep_ref, ww_ref,
                 bw_ref, s_ref):
    _, wpre = _wpre(k_ref, xp_ref, q_ref, sp_ref, gp_ref, bep_ref, ww_ref,
                    bw_ref)
    _stats_update(s_ref, wpre, pl.program_id(0) == 0)


def _wstats(k3, xp3, q, sp, gp, bep, ww, bw):
    return pl.pallas_call(
        _wstats_body,
        grid=(NBLK,),
        in_specs=[
            pl.BlockSpec((K, BQ, C), lambda i: (0, i, 0)),
            pl.BlockSpec((K, BQ, C), lambda i: (0, i, 0)),
            pl.BlockSpec((BQ, C), lambda i: (i, 0)),
            pl.BlockSpec((8, C), lambda i: (0, 0)),
            pl.BlockSpec((1, C), lambda i: (0, 0)),
            pl.BlockSpec((1, C), lambda i: (0, 0)),
            pl.BlockSpec((C, C), lambda i: (0, 0)),
            pl.BlockSpec((1, C), lambda i: (0, 0)),
        ],
        out_specs=pl.BlockSpec((8, C), lambda i: (0, 0)),
        out_shape=jax.ShapeDtypeStruct((8, C), jnp.float32),
    )(k3, xp3, q, sp, gp, bep, ww, bw)


def _att_body(k_ref, v_ref, xp_ref, q_ref, sp_ref, gp_ref, bep_ref, ww_ref,
              bw_ref, sw_ref, gw_ref, bew_ref, att_ref, s_ref):
    pf, wpre = _wpre(k_ref, xp_ref, q_ref, sp_ref, gp_ref, bep_ref, ww_ref,
                     bw_ref)
    sc, sh = _affine(sw_ref, float(NK), gw_ref[...], bew_ref[...])
    wf = jnp.maximum(wpre * sc + sh, 0.0).reshape(K, BQ, C)
    m = jnp.max(wf, axis=0, keepdims=True)
    e = jnp.exp(wf - m)
    den = jnp.sum(e, axis=0, keepdims=True)
    w3 = e / den
    att = jnp.sum((v_ref[...] + pf.reshape(K, BQ, C)) * w3, axis=0)
    att_ref[...] = att
    _stats_update(s_ref, att, pl.program_id(0) == 0)


def _att(k3, v3, xp3, q, sp, gp, bep, ww, bw, sw, gw, bew):
    return pl.pallas_call(
        _att_body,
        grid=(NBLK,),
        in_specs=[
            pl.BlockSpec((K, BQ, C), lambda i: (0, i, 0)),
            pl.BlockSpec((K, BQ, C), lambda i: (0, i, 0)),
            pl.BlockSpec((K, BQ, C), lambda i: (0, i, 0)),
            pl.BlockSpec((BQ, C), lambda i: (i, 0)),
            pl.BlockSpec((8, C), lambda i: (0, 0)),
            pl.BlockSpec((1, C), lambda i: (0, 0)),
            pl.BlockSpec((1, C), lambda i: (0, 0)),
            pl.BlockSpec((C, C), lambda i: (0, 0)),
            pl.BlockSpec((1, C), lambda i: (0, 0)),
            pl.BlockSpec((8, C), lambda i: (0, 0)),
            pl.BlockSpec((1, C), lambda i: (0, 0)),
            pl.BlockSpec((1, C), lambda i: (0, 0)),
        ],
        out_specs=[
            pl.BlockSpec((BQ, C), lambda i: (i, 0)),
            pl.BlockSpec((8, C), lambda i: (0, 0)),
        ],
        out_shape=[
            jax.ShapeDtypeStruct((N, C), jnp.float32),
            jax.ShapeDtypeStruct((8, C), jnp.float32),
        ],
    )(k3, v3, xp3, q, sp, gp, bep, ww, bw, sw, gw, bew)


def _final_body(att_ref, f_ref, so_ref, go_ref, beo_ref, o_ref):
    sc, sh = _affine(so_ref, float(N), go_ref[...], beo_ref[...])
    o_ref[...] = jnp.maximum(att_ref[...] * sc + sh + f_ref[...], 0.0)


def _final(att, feats, so, go, beo):
    return pl.pallas_call(
        _final_body,
        out_shape=jax.ShapeDtypeStruct((N, C), jnp.float32),
    )(att, feats, so, go, beo)


def kernel(points, neighbors, feats, Wq, bq, Wk, bk, Wv, bv, Wp, bp,
           g_p, be_p, Ww, bw, g_w, be_w, g_o, be_o):
    row = lambda x: x.reshape(1, C)
    p16 = jnp.pad(points, ((0, 0), (0, PW - 3)))
    wp16 = jnp.pad(Wp, ((0, PW - 3), (0, 0)))
    idx = jnp.pad(neighbors.T.reshape(1, NK),
                  ((0, 0), (0, NKPAD - NK)))

    q, k, v, xp_table = _qkv(feats, p16, Wq, row(bq), Wk, row(bk),
                             Wv, row(bv), wp16, row(bp))
    xp_nb = _sc_gather(xp_table, idx)[:NK].reshape(K, N, C)
    k_nb = _sc_gather(k, idx)[:NK].reshape(K, N, C)
    v_nb = _sc_gather(v, idx)[:NK].reshape(K, N, C)

    sp = _pstats(xp_nb)
    sw = _wstats(k_nb, xp_nb, q, sp, row(g_p), row(be_p), Ww, row(bw))
    att, so = _att(k_nb, v_nb, xp_nb, q, sp, row(g_p), row(be_p),
                   Ww, row(bw), sw, row(g_w), row(be_w))
    return _final(att, feats, so, row(g_o), row(be_o))


# speedup vs baseline: 1.6298x; 1.4545x over previous
import functools

import jax
import jax.numpy as jnp
from jax.experimental import pallas as pl
from jax.experimental.pallas import tpu as pltpu
from jax.experimental.pallas import tpu_sc as plsc

N = 10000
K = 32
C = 128
C3 = 3 * C
NK = N * K
PW = 16
BQ = 200
NBLK = N // BQ
GATHER_WIN = 128
NKPAD = 327680
EPS = 1e-5


def _sc_gather(table, idx):
    mesh = plsc.VectorSubcoreMesh(core_axis_name="c", subcore_axis_name="s")

    @functools.partial(
        pl.kernel,
        out_type=jax.ShapeDtypeStruct((NKPAD, C3), table.dtype),
        mesh=mesh,
    )
    def knl(tab_hbm, idx_hbm, out_hbm):
        def body(i_vmem, o_vmem):
            pltpu.sync_copy(tab_hbm.at[i_vmem.at[0]], o_vmem)

        pltpu.emit_pipeline(
            body,
            grid=(NKPAD // GATHER_WIN,),
            in_specs=[pl.BlockSpec((1, GATHER_WIN), lambda i: (0, i))],
            out_specs=[pl.BlockSpec((GATHER_WIN, C3), lambda i: (i, 0))],
            core_axis_name=("c", "s"),
            dimension_semantics=(pltpu.PARALLEL,),
        )(idx_hbm, out_hbm)

    return knl(table, idx)


def _qkv_body(f_ref, p_ref, wq_ref, bq_ref, wk_ref, bk_ref, wv_ref, bv_ref,
              wp_ref, bp_ref, q_ref, tab_ref):
    f = f_ref[...]
    q_ref[...] = jnp.dot(f, wq_ref[...], preferred_element_type=jnp.float32) + bq_ref[...]
    k = jnp.dot(f, wk_ref[...], preferred_element_type=jnp.float32) + bk_ref[...]
    v = jnp.dot(f, wv_ref[...], preferred_element_type=jnp.float32) + bv_ref[...]
    xp = jnp.dot(p_ref[...], wp_ref[...],
                 preferred_element_type=jnp.float32) + bp_ref[...]
    tab_ref[...] = jnp.concatenate([xp, k, v], axis=1)


def _qkv(feats, p16, wq, bq, wk, bk, wv, bv, wp16, bp):
    return pl.pallas_call(
        _qkv_body,
        out_shape=(jax.ShapeDtypeStruct((N, C), jnp.float32),
                   jax.ShapeDtypeStruct((N, C3), jnp.float32)),
    )(feats, p16, wq, bq, wk, bk, wv, bv, wp16, bp)


def _stats_update(s_ref, x, first):
    upd = jnp.concatenate(
        [jnp.sum(x, axis=0, keepdims=True),
         jnp.sum(x * x, axis=0, keepdims=True),
         jnp.zeros((6, C), jnp.float32)], axis=0)

    @pl.when(first)
    def _():
        s_ref[...] = jnp.zeros_like(s_ref)

    s_ref[...] += upd


def _affine(s_ref, count, g, be):
    mean = s_ref[0:1, :] / count
    var = s_ref[1:2, :] / count - mean * mean
    scale = jax.lax.rsqrt(var + EPS) * g
    return scale, be - mean * scale


def _pstats_body(g_ref, s_ref):
    x = g_ref[:, :, 0:C].astype(jnp.float32).reshape(K * BQ, C)
    _stats_update(s_ref, x, pl.program_id(0) == 0)


def _pstats(g3):
    return pl.pallas_call(
        _pstats_body,
        grid=(NBLK,),
        in_specs=[pl.BlockSpec((K, BQ, C3), lambda i: (0, i, 0))],
        out_specs=pl.BlockSpec((8, C), lambda i: (0, 0)),
        out_shape=jax.ShapeDtypeStruct((8, C), jnp.float32),
    )(g3)


def _wpre(g_ref, q_ref, sp_ref, gp_ref, bep_ref, ww_ref, bw_ref):
    sc, sh = _affine(sp_ref, float(NK), gp_ref[...], bep_ref[...])
    xp = g_ref[:, :, 0:C].astype(jnp.float32).reshape(K * BQ, C)
    k = g_ref[:, :, C:2 * C].astype(jnp.float32)
    pf = jnp.maximum(xp * sc + sh, 0.0)
    kq = (k * q_ref[...][None]).reshape(K * BQ, C)
    wpre = jnp.dot(kq + pf, ww_ref[...],
                   preferred_element_type=jnp.float32) + bw_ref[...]
    return pf, wpre


def _wstats_body(g_ref, q_ref, sp_ref, gp_ref, bep_ref, ww_ref, bw_ref,
                 s_ref):
    _, wpre = _wpre(g_ref, q_ref, sp_ref, gp_ref, bep_ref, ww_ref, bw_ref)
    _stats_update(s_ref, wpre, pl.program_id(0) == 0)


def _wstats(g3, q, sp, gp, bep, ww, bw):
    return pl.pallas_call(
        _wstats_body,
        grid=(NBLK,),
        in_specs=[
            pl.BlockSpec((K, BQ, C3), lambda i: (0, i, 0)),
            pl.BlockSpec((BQ, C), lambda i: (i, 0)),
            pl.BlockSpec((8, C), lambda i: (0, 0)),
            pl.BlockSpec((1, C), lambda i: (0, 0)),
            pl.BlockSpec((1, C), lambda i: (0, 0)),
            pl.BlockSpec((C, C), lambda i: (0, 0)),
            pl.BlockSpec((1, C), lambda i: (0, 0)),
        ],
        out_specs=pl.BlockSpec((8, C), lambda i: (0, 0)),
        out_shape=jax.ShapeDtypeStruct((8, C), jnp.float32),
    )(g3, q, sp, gp, bep, ww, bw)


def _att_body(g_ref, q_ref, sp_ref, gp_ref, bep_ref, ww_ref, bw_ref,
              sw_ref, gw_ref, bew_ref, att_ref, s_ref):
    pf, wpre = _wpre(g_ref, q_ref, sp_ref, gp_ref, bep_ref, ww_ref, bw_ref)
    sc, sh = _affine(sw_ref, float(NK), gw_ref[...], bew_ref[...])
    wf = jnp.maximum(wpre * sc + sh, 0.0).reshape(K, BQ, C)
    m = jnp.max(wf, axis=0, keepdims=True)
    e = jnp.exp(wf - m)
    den = jnp.sum(e, axis=0, keepdims=True)
    w3 = e / den
    v = g_ref[:, :, 2 * C:C3].astype(jnp.float32)
    att = jnp.sum((v + pf.reshape(K, BQ, C)) * w3, axis=0)
    att_ref[...] = att
    _stats_update(s_ref, att, pl.program_id(0) == 0)


def _att(g3, q, sp, gp, bep, ww, bw, sw, gw, bew):
    return pl.pallas_call(
        _att_body,
        grid=(NBLK,),
        in_specs=[
            pl.BlockSpec((K, BQ, C3), lambda i: (0, i, 0)),
            pl.BlockSpec((BQ, C), lambda i: (i, 0)),
            pl.BlockSpec((8, C), lambda i: (0, 0)),
            pl.BlockSpec((1, C), lambda i: (0, 0)),
            pl.BlockSpec((1, C), lambda i: (0, 0)),
            pl.BlockSpec((C, C), lambda i: (0, 0)),
            pl.BlockSpec((1, C), lambda i: (0, 0)),
            pl.BlockSpec((8, C), lambda i: (0, 0)),
            pl.BlockSpec((1, C), lambda i: (0, 0)),
            pl.BlockSpec((1, C), lambda i: (0, 0)),
        ],
        out_specs=[
            pl.BlockSpec((BQ, C), lambda i: (i, 0)),
            pl.BlockSpec((8, C), lambda i: (0, 0)),
        ],
        out_shape=[
            jax.ShapeDtypeStruct((N, C), jnp.float32),
            jax.ShapeDtypeStruct((8, C), jnp.float32),
        ],
    )(g3, q, sp, gp, bep, ww, bw, sw, gw, bew)


def _final_body(att_ref, f_ref, so_ref, go_ref, beo_ref, o_ref):
    sc, sh = _affine(so_ref, float(N), go_ref[...], beo_ref[...])
    o_ref[...] = jnp.maximum(att_ref[...] * sc + sh + f_ref[...], 0.0)


def _final(att, feats, so, go, beo):
    return pl.pallas_call(
        _final_body,
        out_shape=jax.ShapeDtypeStruct((N, C), jnp.float32),
    )(att, feats, so, go, beo)


def kernel(points, neighbors, feats, Wq, bq, Wk, bk, Wv, bv, Wp, bp,
           g_p, be_p, Ww, bw, g_w, be_w, g_o, be_o):
    row = lambda x: x.reshape(1, C)
    p16 = jnp.pad(points, ((0, 0), (0, PW - 3)))
    wp16 = jnp.pad(Wp, ((0, PW - 3), (0, 0)))
    idx = jnp.pad(neighbors.T.reshape(1, NK),
                  ((0, 0), (0, NKPAD - NK)))

    q, table = _qkv(feats, p16, Wq, row(bq), Wk, row(bk), Wv, row(bv),
                    wp16, row(bp))
    g3 = _sc_gather(table, idx)[:NK].reshape(K, N, C3)

    sp = _pstats(g3)
    sw = _wstats(g3, q, sp, row(g_p), row(be_p), Ww, row(bw))
    att, so = _att(g3, q, sp, row(g_p), row(be_p), Ww, row(bw),
                   sw, row(g_w), row(be_w))
    return _final(att, feats, so, row(g_o), row(be_o))


# speedup vs baseline: 2.0589x; 1.2633x over previous
import functools

import jax
import jax.numpy as jnp
from jax.experimental import pallas as pl
from jax.experimental.pallas import tpu as pltpu
from jax.experimental.pallas import tpu_sc as plsc

N = 10000
K = 32
C = 128
C2 = 2 * C
NK = N * K
PW = 16
BQ = 200
NBLK = N // BQ
GATHER_WIN = 128
NKPAD = 327680
EPS = 1e-5

_HI = -65536


def _f2i(x):
    return jax.lax.bitcast_convert_type(x, jnp.int32)


def _i2f(x):
    return jax.lax.bitcast_convert_type(x, jnp.float32)


def _sc_gather(table, idx):
    mesh = plsc.VectorSubcoreMesh(core_axis_name="c", subcore_axis_name="s")

    @functools.partial(
        pl.kernel,
        out_type=jax.ShapeDtypeStruct((NKPAD, C2), table.dtype),
        mesh=mesh,
    )
    def knl(tab_hbm, idx_hbm, out_hbm):
        def body(i_vmem, o_vmem):
            pltpu.sync_copy(tab_hbm.at[i_vmem.at[0]], o_vmem)

        pltpu.emit_pipeline(
            body,
            grid=(NKPAD // GATHER_WIN,),
            in_specs=[pl.BlockSpec((1, GATHER_WIN), lambda i: (0, i))],
            out_specs=[pl.BlockSpec((GATHER_WIN, C2), lambda i: (i, 0))],
            core_axis_name=("c", "s"),
            dimension_semantics=(pltpu.PARALLEL,),
        )(idx_hbm, out_hbm)

    return knl(table, idx)


def _qkv_body(f_ref, p_ref, wq_ref, bq_ref, wk_ref, bk_ref, wv_ref, bv_ref,
              wp_ref, bp_ref, q_ref, tab_ref):
    f = f_ref[...]
    q_ref[...] = jnp.dot(f, wq_ref[...], preferred_element_type=jnp.float32) + bq_ref[...]
    k = jnp.dot(f, wk_ref[...], preferred_element_type=jnp.float32) + bk_ref[...]
    v = jnp.dot(f, wv_ref[...], preferred_element_type=jnp.float32) + bv_ref[...]
    xp = jnp.dot(p_ref[...], wp_ref[...],
                 preferred_element_type=jnp.float32) + bp_ref[...]
    kb = _f2i(k.astype(jnp.bfloat16).astype(jnp.float32))
    vb = _f2i(v.astype(jnp.bfloat16).astype(jnp.float32))
    kv = _i2f((vb & _HI) | jax.lax.shift_right_logical(kb, 16))
    tab_ref[...] = jnp.concatenate([kv, xp], axis=1)


def _qkv(feats, p16, wq, bq, wk, bk, wv, bv, wp16, bp):
    return pl.pallas_call(
        _qkv_body,
        out_shape=(jax.ShapeDtypeStruct((N, C), jnp.float32),
                   jax.ShapeDtypeStruct((N, C2), jnp.float32)),
    )(feats, p16, wq, bq, wk, bk, wv, bv, wp16, bp)


def _unpack_k(g):
    return _i2f(jax.lax.shift_left(_f2i(g[:, :, 0:C]), 16))


def _unpack_v(g):
    return _i2f(_f2i(g[:, :, 0:C]) & _HI)


def _stats_update(s_ref, x, first):
    upd = jnp.concatenate(
        [jnp.sum(x, axis=0, keepdims=True),
         jnp.sum(x * x, axis=0, keepdims=True),
         jnp.zeros((6, C), jnp.float32)], axis=0)

    @pl.when(first)
    def _():
        s_ref[...] = jnp.zeros_like(s_ref)

    s_ref[...] += upd


def _affine(s_ref, count, g, be):
    mean = s_ref[0:1, :] / count
    var = s_ref[1:2, :] / count - mean * mean
    scale = jax.lax.rsqrt(var + EPS) * g
    return scale, be - mean * scale


def _pstats_body(g_ref, s_ref):
    x = g_ref[:, :, C:C2].reshape(K * BQ, C)
    _stats_update(s_ref, x, pl.program_id(0) == 0)


def _pstats(g3):
    return pl.pallas_call(
        _pstats_body,
        grid=(NBLK,),
        in_specs=[pl.BlockSpec((K, BQ, C2), lambda i: (0, i, 0))],
        out_specs=pl.BlockSpec((8, C), lambda i: (0, 0)),
        out_shape=jax.ShapeDtypeStruct((8, C), jnp.float32),
    )(g3)


def _wpre(g_ref, q_ref, sp_ref, gp_ref, bep_ref, ww_ref, bw_ref):
    g = g_ref[...]
    sc, sh = _affine(sp_ref, float(NK), gp_ref[...], bep_ref[...])
    xp = g[:, :, C:C2].reshape(K * BQ, C)
    pf = jnp.maximum(xp * sc + sh, 0.0)
    kq = (_unpack_k(g) * q_ref[...][None]).reshape(K * BQ, C)
    wpre = jnp.dot(kq + pf, ww_ref[...],
                   preferred_element_type=jnp.float32) + bw_ref[...]
    return pf, wpre


def _wstats_body(g_ref, q_ref, sp_ref, gp_ref, bep_ref, ww_ref, bw_ref,
                 s_ref):
    _, wpre = _wpre(g_ref, q_ref, sp_ref, gp_ref, bep_ref, ww_ref, bw_ref)
    _stats_update(s_ref, wpre, pl.program_id(0) == 0)


def _wstats(g3, q, sp, gp, bep, ww, bw):
    return pl.pallas_call(
        _wstats_body,
        grid=(NBLK,),
        in_specs=[
            pl.BlockSpec((K, BQ, C2), lambda i: (0, i, 0)),
            pl.BlockSpec((BQ, C), lambda i: (i, 0)),
            pl.BlockSpec((8, C), lambda i: (0, 0)),
            pl.BlockSpec((1, C), lambda i: (0, 0)),
            pl.BlockSpec((1, C), lambda i: (0, 0)),
            pl.BlockSpec((C, C), lambda i: (0, 0)),
            pl.BlockSpec((1, C), lambda i: (0, 0)),
        ],
        out_specs=pl.BlockSpec((8, C), lambda i: (0, 0)),
        out_shape=jax.ShapeDtypeStruct((8, C), jnp.float32),
    )(g3, q, sp, gp, bep, ww, bw)


def _att_body(g_ref, q_ref, sp_ref, gp_ref, bep_ref, ww_ref, bw_ref,
              sw_ref, gw_ref, bew_ref, att_ref, s_ref):
    pf, wpre = _wpre(g_ref, q_ref, sp_ref, gp_ref, bep_ref, ww_ref, bw_ref)
    sc, sh = _affine(sw_ref, float(NK), gw_ref[...], bew_ref[...])
    wf = jnp.maximum(wpre * sc + sh, 0.0).reshape(K, BQ, C)
    m = jnp.max(wf, axis=0, keepdims=True)
    e = jnp.exp(wf - m)
    den = jnp.sum(e, axis=0, keepdims=True)
    w3 = e / den
    att = jnp.sum((_unpack_v(g_ref[...]) + pf.reshape(K, BQ, C)) * w3, axis=0)
    att_ref[...] = att
    _stats_update(s_ref, att, pl.program_id(0) == 0)


def _att(g3, q, sp, gp, bep, ww, bw, sw, gw, bew):
    return pl.pallas_call(
        _att_body,
        grid=(NBLK,),
        in_specs=[
            pl.BlockSpec((K, BQ, C2), lambda i: (0, i, 0)),
            pl.BlockSpec((BQ, C), lambda i: (i, 0)),
            pl.BlockSpec((8, C), lambda i: (0, 0)),
            pl.BlockSpec((1, C), lambda i: (0, 0)),
            pl.BlockSpec((1, C), lambda i: (0, 0)),
            pl.BlockSpec((C, C), lambda i: (0, 0)),
            pl.BlockSpec((1, C), lambda i: (0, 0)),
            pl.BlockSpec((8, C), lambda i: (0, 0)),
            pl.BlockSpec((1, C), lambda i: (0, 0)),
            pl.BlockSpec((1, C), lambda i: (0, 0)),
        ],
        out_specs=[
            pl.BlockSpec((BQ, C), lambda i: (i, 0)),
            pl.BlockSpec((8, C), lambda i: (0, 0)),
        ],
        out_shape=[
            jax.ShapeDtypeStruct((N, C), jnp.float32),
            jax.ShapeDtypeStruct((8, C), jnp.float32),
        ],
    )(g3, q, sp, gp, bep, ww, bw, sw, gw, bew)


def _final_body(att_ref, f_ref, so_ref, go_ref, beo_ref, o_ref):
    sc, sh = _affine(so_ref, float(N), go_ref[...], beo_ref[...])
    o_ref[...] = jnp.maximum(att_ref[...] * sc + sh + f_ref[...], 0.0)


def _final(att, feats, so, go, beo):
    return pl.pallas_call(
        _final_body,
        out_shape=jax.ShapeDtypeStruct((N, C), jnp.float32),
    )(att, feats, so, go, beo)


def kernel(points, neighbors, feats, Wq, bq, Wk, bk, Wv, bv, Wp, bp,
           g_p, be_p, Ww, bw, g_w, be_w, g_o, be_o):
    row = lambda x: x.reshape(1, C)
    p16 = jnp.pad(points, ((0, 0), (0, PW - 3)))
    wp16 = jnp.pad(Wp, ((0, PW - 3), (0, 0)))
    idx = jnp.pad(neighbors.T.reshape(1, NK),
                  ((0, 0), (0, NKPAD - NK)))

    q, table = _qkv(feats, p16, Wq, row(bq), Wk, row(bk), Wv, row(bv),
                    wp16, row(bp))
    g3 = _sc_gather(table, idx)[:NK].reshape(K, N, C2)

    sp = _pstats(g3)
    sw = _wstats(g3, q, sp, row(g_p), row(be_p), Ww, row(bw))
    att, so = _att(g3, q, sp, row(g_p), row(be_p), Ww, row(bw),
                   sw, row(g_w), row(be_w))
    return _final(att, feats, so, row(g_o), row(be_o))
